# baseline (speedup 1.0000x reference)
import jax
import jax.numpy as jnp
from jax.experimental import pallas as pl
from jax.experimental.pallas import tpu as pltpu

_B, _P, _C = 64, 8732, 21
_EPS = 1e-7


def _body(conf_ref, shuf_ref, interp_ref, out_ref, acc_ref):
    b = pl.program_id(0)
    first = b == 0
    last = b == _B - 1

    @pl.when(first)
    def _init():
        acc_ref[0] = 0.0
        acc_ref[1] = 0.0
        acc_ref[2] = 0.0
        acc_ref[3] = 0.0

    x = conf_ref[0]
    s = shuf_ref[0]
    i = interp_ref[0]

    x0 = x[:, 0]
    s0 = s[:, 0]
    lm = jnp.max(x, axis=1) > x0
    rm = jnp.max(s, axis=1) > s0
    ol = jnp.logical_and(lm, jnp.logical_not(rm))
    orr = jnp.logical_and(rm, jnp.logical_not(lm))

    ri = 1.0 / (i + _EPS)
    t = x + _EPS
    ts = s + _EPS
    kl_l = t * jnp.log(t * ri)
    kl_r = ts * jnp.log(ts * ri)

    olf = ol.astype(jnp.float32)[:, None]
    orf = orr.astype(jnp.float32)[:, None]
    acc_ref[0] += jnp.sum(kl_l * olf)
    acc_ref[1] += jnp.sum(olf)
    acc_ref[2] += jnp.sum(kl_r * orf)
    acc_ref[3] += jnp.sum(orf)

    @pl.when(last)
    def _fin():
        sl, cl, sr, cr = acc_ref[0], acc_ref[1], acc_ref[2], acc_ref[3]
        loss_l = jnp.where(cl > 0.0, sl / jnp.maximum(cl, 1.0), 0.0)
        loss_r = jnp.where(cr > 0.0, sr / jnp.maximum(cr, 1.0), 0.0)
        out_ref[0] = loss_l + loss_r


def kernel(args, lam, conf, conf_flip, loc, loc_flip, conf_shuffle,
           conf_interpolation, loc_shuffle, loc_interpolation):
    half = _B // 2
    loss = pl.pallas_call(
        _body,
        grid=(_B,),
        in_specs=[
            pl.BlockSpec((1, _P, _C), lambda b: (b, 0, 0)),
            pl.BlockSpec((1, _P, _C), lambda b: ((b + half) % _B, 0, 0)),
            pl.BlockSpec((1, _P, _C), lambda b: (b, 0, 0)),
        ],
        out_specs=pl.BlockSpec(memory_space=pltpu.SMEM),
        out_shape=jax.ShapeDtypeStruct((1,), jnp.float32),
        scratch_shapes=[pltpu.SMEM((4,), jnp.float32)],
    )(conf, conf_shuffle, conf_interpolation)
    return (jnp.zeros((1,), jnp.float32), loss[0])
